# baseline (speedup 1.0000x reference)
import functools

import jax
import jax.numpy as jnp
from jax import lax
from jax.experimental import pallas as pl
from jax.experimental.pallas import tpu as pltpu
from jax.experimental.pallas import tpu_sc as plsc

N = 10000
E = 320000
D = 128
NC = 2
NS = 16
NW = NC * NS
CHUNK = 125
NCHUNK = (E // NW) // CHUNK
NBUF = 4
N_PAD = 10240
ROWS_SUB = N_PAD // NS
ZROWS = 128
DH = D // 2
DEG_PAD = 10240
DEG_SUB = DEG_PAD // NS
RB = 1000
G = N // RB
EPS = 1e-5

def _sc_deg_body(dstr, out, dst_v, ones_v, zb, acc, sem):
    del sem
    cid = lax.axis_index("c")
    sid = lax.axis_index("s")
    wid = sid * NC + cid

    @pl.loop(0, CHUNK)
    def _(i):
        ones_v[i, :] = jnp.ones((16,), jnp.float32)

    @pl.loop(0, DEG_SUB)
    def _(i):
        zb[i, :] = jnp.zeros((16,), jnp.float32)

    pltpu.sync_copy(dstr.at[wid], dst_v)
    pltpu.sync_copy(zb, acc.at[pl.ds(sid * DEG_SUB, DEG_SUB)])
    plsc.subcore_barrier()

    @pl.loop(0, NCHUNK)
    def _(j):
        pltpu.sync_copy(ones_v, acc.at[dst_v.at[j]], add=True)

    plsc.subcore_barrier()
    pltpu.sync_copy(acc.at[pl.ds(sid * DEG_SUB, DEG_SUB)],
                    out.at[cid, pl.ds(sid * DEG_SUB, DEG_SUB)])


def _sc_edge_body(hs_lo, hs_hi, srcr, dstr, out,
                  src_v, dst_v, rows, zbuf, acc, gsems, ssems):
    cid = lax.axis_index("c")
    sid = lax.axis_index("s")
    wid = sid * NC + cid

    @pl.loop(0, ZROWS)
    def _(i):
        @pl.loop(0, DH // 16)
        def _(j):
            zbuf[i, pl.ds(j * 16, 16)] = jnp.zeros((16,), jnp.float32)

    pltpu.sync_copy(srcr.at[wid], src_v)
    pltpu.sync_copy(dstr.at[wid], dst_v)

    for half, tbl in enumerate((hs_lo, hs_hi)):
        for k in range(ROWS_SUB // ZROWS):
            pltpu.sync_copy(
                zbuf, acc.at[pl.ds(sid * ROWS_SUB + k * ZROWS, ZROWS)])
        plsc.subcore_barrier()

        for b in range(NBUF):
            pltpu.async_copy(tbl.at[src_v.at[b]], rows[b], gsems[b])

        @pl.loop(0, NCHUNK, step=NBUF)
        def _(j):
            for b in range(NBUF):
                pltpu.make_async_copy(
                    tbl.at[src_v.at[j + b]], rows[b], gsems[b]).wait()
                pltpu.async_copy(
                    rows[b], acc.at[dst_v.at[j + b]], ssems[b], add=True)
            for b in range(NBUF):
                pltpu.make_async_copy(
                    rows[b], acc.at[dst_v.at[j + b]], ssems[b]).wait()

                @pl.when(j + b + NBUF < NCHUNK)
                def _():
                    pltpu.async_copy(
                        tbl.at[src_v.at[j + b + NBUF]], rows[b], gsems[b])

        plsc.subcore_barrier()
        pltpu.sync_copy(acc.at[pl.ds(sid * ROWS_SUB, ROWS_SUB)],
                        out.at[half, cid, pl.ds(sid * ROWS_SUB, ROWS_SUB)])
        plsc.subcore_barrier()


@functools.lru_cache(maxsize=None)
def _sc_kernels():
    mesh = plsc.VectorSubcoreMesh(
        core_axis_name="c", subcore_axis_name="s",
        num_cores=NC, num_subcores=NS)
    sc_deg = pl.kernel(
        _sc_deg_body,
        out_type=jax.ShapeDtypeStruct((NC, DEG_PAD, 16), jnp.float32),
        mesh=mesh,
        compiler_params=pltpu.CompilerParams(use_tc_tiling_on_sc=False),
        scratch_types=[
            pltpu.VMEM((NCHUNK, CHUNK), jnp.int32),
            pltpu.VMEM((CHUNK, 16), jnp.float32),
            pltpu.VMEM((DEG_SUB, 16), jnp.float32),
            pltpu.VMEM_SHARED((DEG_PAD, 16), jnp.float32),
            pltpu.SemaphoreType.DMA,
        ],
    )
    sc_edge = pl.kernel(
        _sc_edge_body,
        out_type=jax.ShapeDtypeStruct((2, NC, N_PAD, DH), jnp.float32),
        mesh=mesh,
        compiler_params=pltpu.CompilerParams(use_tc_tiling_on_sc=False),
        scratch_types=[
            pltpu.VMEM((NCHUNK, CHUNK), jnp.int32),
            pltpu.VMEM((NCHUNK, CHUNK), jnp.int32),
            [pltpu.VMEM((CHUNK, DH), jnp.float32) for _ in range(NBUF)],
            pltpu.VMEM((ZROWS, DH), jnp.float32),
            pltpu.VMEM_SHARED((N_PAD, DH), jnp.float32),
            [pltpu.SemaphoreType.DMA for _ in range(NBUF)],
            [pltpu.SemaphoreType.DMA for _ in range(NBUF)],
        ],
    )
    return sc_deg, sc_edge


def _dotT(a, w):
    return lax.dot_general(a, w, (((1,), (1,)), ((), ())),
                           preferred_element_type=jnp.float32)


def _tc1_body(x_ref, d0_ref, d1_ref, w0_ref, p_ref,
              hsl_ref, hsh_ref, res_ref, dinv_ref):
    deg = d0_ref[...] + d1_ref[...] + 1.0
    dinv = lax.rsqrt(deg)
    xb = x_ref[...]
    hs = _dotT(xb, w0_ref[...]) * dinv
    hsl_ref[...] = hs[:, :DH]
    hsh_ref[...] = hs[:, DH:]
    res_ref[...] = _dotT(xb, p_ref[...])
    dinv_ref[...] = dinv


def _tc1(x, d0, d1, w0, p):
    return pl.pallas_call(
        _tc1_body,
        grid=(G,),
        in_specs=[
            pl.BlockSpec((RB, D), lambda i: (i, 0)),
            pl.BlockSpec((RB, 1), lambda i: (i, 0)),
            pl.BlockSpec((RB, 1), lambda i: (i, 0)),
            pl.BlockSpec((D, D), lambda i: (0, 0)),
            pl.BlockSpec((D, D), lambda i: (0, 0)),
        ],
        out_specs=[
            pl.BlockSpec((RB, DH), lambda i: (i, 0)),
            pl.BlockSpec((RB, DH), lambda i: (i, 0)),
            pl.BlockSpec((RB, D), lambda i: (i, 0)),
            pl.BlockSpec((RB, 1), lambda i: (i, 0)),
        ],
        out_shape=[
            jax.ShapeDtypeStruct((N, DH), jnp.float32),
            jax.ShapeDtypeStruct((N, DH), jnp.float32),
            jax.ShapeDtypeStruct((N, D), jnp.float32),
            jax.ShapeDtypeStruct((N, 1), jnp.float32),
        ],
    )(x, d0, d1, w0, p)


def _norm_apply(c, stv, gw, gb, gms):
    mean = stv[0:1] / N
    ex2 = stv[1:2] / N
    var = ex2 - (2.0 * gms - gms * gms) * mean * mean
    inv = lax.rsqrt(var + EPS)
    return gw * (c - gms * mean) * inv + gb


def _conv_phase(p0l, p0h, p1l, p1h, hsl, hsh, dinv, b, i, conv_s, st_s):
    left = p0l[...] + p1l[...] + hsl[...]
    right = p0h[...] + p1h[...] + hsh[...]
    c = jnp.concatenate([left, right], axis=1) * dinv[...] + b[...]
    conv_s[pl.ds(i * RB, RB), :] = c
    st = jnp.concatenate(
        [jnp.sum(c, axis=0, keepdims=True),
         jnp.sum(c * c, axis=0, keepdims=True)], axis=0)

    @pl.when(i == 0)
    def _():
        st_s[...] = st

    @pl.when(i != 0)
    def _():
        st_s[...] = st_s[...] + st


def _tcA_body(p0l, p0h, p1l, p1h, hsl, hsh, dinv, b, res0, gw, gb, gms, w1,
              h1_ref, hsl1_ref, hsh1_ref, conv_s, st_s):
    p = pl.program_id(0)
    i = pl.program_id(1)

    @pl.when(p == 0)
    def _():
        _conv_phase(p0l, p0h, p1l, p1h, hsl, hsh, dinv, b, i, conv_s, st_s)

    @pl.when(p == 1)
    def _():
        c = conv_s[pl.ds(i * RB, RB), :]
        normed = _norm_apply(c, st_s[...], gw[...], gb[...], gms[...])
        h1 = jnp.maximum(normed, 0.0) + res0[...]
        h1_ref[...] = h1
        hs1 = _dotT(h1, w1[...]) * dinv[...]
        hsl1_ref[...] = hs1[:, :DH]
        hsh1_ref[...] = hs1[:, DH:]


def _tcA(p0l, p0h, p1l, p1h, hsl, hsh, dinv, b, res0, gw, gb, gms, w1):
    half = pl.BlockSpec((RB, DH), lambda p, i: (i, 0))
    row = pl.BlockSpec((RB, D), lambda p, i: (i, 0))
    vec = pl.BlockSpec((1, D), lambda p, i: (0, 0))
    full = pl.BlockSpec((D, D), lambda p, i: (0, 0))
    return pl.pallas_call(
        _tcA_body,
        grid=(2, G),
        in_specs=[half, half, half, half, half, half,
                  pl.BlockSpec((RB, 1), lambda p, i: (i, 0)),
                  vec, row, vec, vec, vec, full],
        out_specs=[row,
                   pl.BlockSpec((RB, DH), lambda p, i: (i, 0)),
                   pl.BlockSpec((RB, DH), lambda p, i: (i, 0))],
        out_shape=[
            jax.ShapeDtypeStruct((N, D), jnp.float32),
            jax.ShapeDtypeStruct((N, DH), jnp.float32),
            jax.ShapeDtypeStruct((N, DH), jnp.float32),
        ],
        scratch_shapes=[
            pltpu.VMEM((N, D), jnp.float32),
            pltpu.VMEM((2, D), jnp.float32),
        ],
    )(p0l, p0h, p1l, p1h, hsl, hsh, dinv, b, res0, gw, gb, gms, w1)


def _tcB_body(p0l, p0h, p1l, p1h, hsl, hsh, dinv, b, h1, gw, gb, gms,
              l0w, l0b, l1w, l1b, tw, tb, out_ref, conv_s, st_s):
    p = pl.program_id(0)
    i = pl.program_id(1)

    @pl.when(p == 0)
    def _():
        _conv_phase(p0l, p0h, p1l, p1h, hsl, hsh, dinv, b, i, conv_s, st_s)

    @pl.when(p == 1)
    def _():
        c = conv_s[pl.ds(i * RB, RB), :]
        normed = _norm_apply(c, st_s[...], gw[...], gb[...], gms[...])
        h2 = jnp.maximum(normed, 0.0) + h1[...]
        m = jnp.maximum(_dotT(h2, l0w[...]) + l0b[...], 0.0)
        m = jnp.maximum(_dotT(m, l1w[...]) + l1b[...], 0.0)
        out_ref[...] = _dotT(m, tw[...]) + tb[...]


def _tcB(p0l, p0h, p1l, p1h, hsl, hsh, dinv, b, h1,
         gw, gb, gms, l0w, l0b, l1w, l1b, tw, tb):
    half = pl.BlockSpec((RB, DH), lambda p, i: (i, 0))
    row = pl.BlockSpec((RB, D), lambda p, i: (i, 0))
    vec = pl.BlockSpec((1, D), lambda p, i: (0, 0))
    full = pl.BlockSpec((D, D), lambda p, i: (0, 0))
    return pl.pallas_call(
        _tcB_body,
        grid=(2, G),
        in_specs=[half, half, half, half, half, half,
                  pl.BlockSpec((RB, 1), lambda p, i: (i, 0)),
                  vec, row, vec, vec, vec,
                  full, vec, full, vec, full, vec],
        out_specs=row,
        out_shape=jax.ShapeDtypeStruct((N, D), jnp.float32),
        scratch_shapes=[
            pltpu.VMEM((N, D), jnp.float32),
            pltpu.VMEM((2, D), jnp.float32),
        ],
    )(p0l, p0h, p1l, p1h, hsl, hsh, dinv, b, h1,
      gw, gb, gms, l0w, l0b, l1w, l1b, tw, tb)


def kernel(x, edge_index, W0, b0, W1, b1, gn0_w, gn0_b, gn0_ms,
           gn1_w, gn1_b, gn1_ms, P, L0_W, L0_b, L1_W, L1_b, T_W, T_b):
    ei = edge_index.astype(jnp.int32)
    srcr = ei[0].reshape(NW, NCHUNK, CHUNK)
    dstr = ei[1].reshape(NW, NCHUNK, CHUNK)

    sc_deg, sc_edge = _sc_kernels()
    degp = sc_deg(dstr)
    d0 = degp[0, :N, 0:1]
    d1 = degp[1, :N, 0:1]

    hsl0, hsh0, res0, dinv = _tc1(x, d0, d1, W0, P)

    pa = sc_edge(hsl0, hsh0, srcr, dstr)
    h1, hsl1, hsh1 = _tcA(pa[0, 0, :N], pa[1, 0, :N], pa[0, 1, :N],
                          pa[1, 1, :N], hsl0, hsh0, dinv, b0.reshape(1, D),
                          res0, gn0_w.reshape(1, D), gn0_b.reshape(1, D),
                          gn0_ms.reshape(1, D), W1)

    pb = sc_edge(hsl1, hsh1, srcr, dstr)
    return _tcB(pb[0, 0, :N], pb[1, 0, :N], pb[0, 1, :N], pb[1, 1, :N],
                hsl1, hsh1, dinv, b1.reshape(1, D), h1,
                gn1_w.reshape(1, D), gn1_b.reshape(1, D),
                gn1_ms.reshape(1, D), L0_W, L0_b.reshape(1, D),
                L1_W, L1_b.reshape(1, D), T_W, T_b.reshape(1, D))


# speedup vs baseline: 1.1474x; 1.1474x over previous
import functools

import jax
import jax.numpy as jnp
from jax import lax
from jax.experimental import pallas as pl
from jax.experimental.pallas import tpu as pltpu
from jax.experimental.pallas import tpu_sc as plsc

N = 10000
E = 320000
D = 128
NC = 2
NS = 16
NW = NC * NS
CHUNK = 125
NCHUNK = (E // NW) // CHUNK
NCHUNK2 = (E // NS) // CHUNK
NBUF = 4
N_PAD = 10240
ROWS_SUB = N_PAD // NS
ZROWS = 128
DH = D // 2
DEG_PAD = 10240
DEG_SUB = DEG_PAD // NS
RB = 1000
G = N // RB
EPS = 1e-5

def _sc_deg_body(dstr, out, dst_v, ones_v, zb, acc, sem):
    del sem
    cid = lax.axis_index("c")
    sid = lax.axis_index("s")
    wid = sid * NC + cid

    @pl.loop(0, CHUNK)
    def _(i):
        ones_v[i, :] = jnp.ones((16,), jnp.float32)

    @pl.loop(0, DEG_SUB)
    def _(i):
        zb[i, :] = jnp.zeros((16,), jnp.float32)

    pltpu.sync_copy(dstr.at[wid], dst_v)
    pltpu.sync_copy(zb, acc.at[pl.ds(sid * DEG_SUB, DEG_SUB)])
    plsc.subcore_barrier()

    @pl.loop(0, NCHUNK)
    def _(j):
        pltpu.sync_copy(ones_v, acc.at[dst_v.at[j]], add=True)

    plsc.subcore_barrier()
    pltpu.sync_copy(acc.at[pl.ds(sid * DEG_SUB, DEG_SUB)],
                    out.at[cid, pl.ds(sid * DEG_SUB, DEG_SUB)])


def _sc_edge_body(hs_lo, hs_hi, srcr, dstr, out,
                  src_v, dst_v, rows, zbuf, acc, gsems, ssems):
    cid = lax.axis_index("c")
    sid = lax.axis_index("s")

    @pl.loop(0, ZROWS)
    def _(i):
        @pl.loop(0, DH // 16)
        def _(j):
            zbuf[i, pl.ds(j * 16, 16)] = jnp.zeros((16,), jnp.float32)

    pltpu.sync_copy(srcr.at[sid], src_v)
    pltpu.sync_copy(dstr.at[sid], dst_v)
    for k in range(ROWS_SUB // ZROWS):
        pltpu.sync_copy(
            zbuf, acc.at[pl.ds(sid * ROWS_SUB + k * ZROWS, ZROWS)])
    plsc.subcore_barrier()

    for half, tbl in enumerate((hs_lo, hs_hi)):
        @pl.when(cid == half)
        def _(tbl=tbl):
            for b in range(NBUF):
                pltpu.async_copy(tbl.at[src_v.at[b]], rows[b], gsems[b])

            @pl.loop(0, NCHUNK2, step=NBUF)
            def _(j):
                for b in range(NBUF):
                    pltpu.make_async_copy(
                        tbl.at[src_v.at[j + b]], rows[b], gsems[b]).wait()
                    pltpu.async_copy(
                        rows[b], acc.at[dst_v.at[j + b]], ssems[b], add=True)
                for b in range(NBUF):
                    pltpu.make_async_copy(
                        rows[b], acc.at[dst_v.at[j + b]], ssems[b]).wait()

                    @pl.when(j + b + NBUF < NCHUNK2)
                    def _():
                        pltpu.async_copy(
                            tbl.at[src_v.at[j + b + NBUF]], rows[b], gsems[b])

    plsc.subcore_barrier()
    pltpu.sync_copy(acc.at[pl.ds(sid * ROWS_SUB, ROWS_SUB)],
                    out.at[cid, pl.ds(sid * ROWS_SUB, ROWS_SUB)])


@functools.lru_cache(maxsize=None)
def _sc_kernels():
    mesh = plsc.VectorSubcoreMesh(
        core_axis_name="c", subcore_axis_name="s",
        num_cores=NC, num_subcores=NS)
    sc_deg = pl.kernel(
        _sc_deg_body,
        out_type=jax.ShapeDtypeStruct((NC, DEG_PAD, 16), jnp.float32),
        mesh=mesh,
        compiler_params=pltpu.CompilerParams(use_tc_tiling_on_sc=False),
        scratch_types=[
            pltpu.VMEM((NCHUNK, CHUNK), jnp.int32),
            pltpu.VMEM((CHUNK, 16), jnp.float32),
            pltpu.VMEM((DEG_SUB, 16), jnp.float32),
            pltpu.VMEM_SHARED((DEG_PAD, 16), jnp.float32),
            pltpu.SemaphoreType.DMA,
        ],
    )
    sc_edge = pl.kernel(
        _sc_edge_body,
        out_type=jax.ShapeDtypeStruct((NC, N_PAD, DH), jnp.float32),
        mesh=mesh,
        compiler_params=pltpu.CompilerParams(use_tc_tiling_on_sc=False),
        scratch_types=[
            pltpu.VMEM((NCHUNK2, CHUNK), jnp.int32),
            pltpu.VMEM((NCHUNK2, CHUNK), jnp.int32),
            [pltpu.VMEM((CHUNK, DH), jnp.float32) for _ in range(NBUF)],
            pltpu.VMEM((ZROWS, DH), jnp.float32),
            pltpu.VMEM_SHARED((N_PAD, DH), jnp.float32),
            [pltpu.SemaphoreType.DMA for _ in range(NBUF)],
            [pltpu.SemaphoreType.DMA for _ in range(NBUF)],
        ],
    )
    return sc_deg, sc_edge


def _dotT(a, w):
    return lax.dot_general(a, w, (((1,), (1,)), ((), ())),
                           preferred_element_type=jnp.float32)


def _tc1_body(x_ref, d0_ref, d1_ref, w0_ref, p_ref,
              hsl_ref, hsh_ref, res_ref, dinv_ref):
    deg = d0_ref[...] + d1_ref[...] + 1.0
    dinv = lax.rsqrt(deg)
    xb = x_ref[...]
    hs = _dotT(xb, w0_ref[...]) * dinv
    hsl_ref[...] = hs[:, :DH]
    hsh_ref[...] = hs[:, DH:]
    res_ref[...] = _dotT(xb, p_ref[...])
    dinv_ref[...] = dinv


def _tc1(x, d0, d1, w0, p):
    return pl.pallas_call(
        _tc1_body,
        grid=(G,),
        in_specs=[
            pl.BlockSpec((RB, D), lambda i: (i, 0)),
            pl.BlockSpec((RB, 1), lambda i: (i, 0)),
            pl.BlockSpec((RB, 1), lambda i: (i, 0)),
            pl.BlockSpec((D, D), lambda i: (0, 0)),
            pl.BlockSpec((D, D), lambda i: (0, 0)),
        ],
        out_specs=[
            pl.BlockSpec((RB, DH), lambda i: (i, 0)),
            pl.BlockSpec((RB, DH), lambda i: (i, 0)),
            pl.BlockSpec((RB, D), lambda i: (i, 0)),
            pl.BlockSpec((RB, 1), lambda i: (i, 0)),
        ],
        out_shape=[
            jax.ShapeDtypeStruct((N, DH), jnp.float32),
            jax.ShapeDtypeStruct((N, DH), jnp.float32),
            jax.ShapeDtypeStruct((N, D), jnp.float32),
            jax.ShapeDtypeStruct((N, 1), jnp.float32),
        ],
    )(x, d0, d1, w0, p)


def _norm_apply(c, stv, gw, gb, gms):
    mean = stv[0:1] / N
    ex2 = stv[1:2] / N
    var = ex2 - (2.0 * gms - gms * gms) * mean * mean
    inv = lax.rsqrt(var + EPS)
    return gw * (c - gms * mean) * inv + gb


def _tc2_body(plo, phi, hsl, hsh, dinv, b, conv_ref, st_ref):
    i = pl.program_id(0)
    left = plo[...] + hsl[...]
    right = phi[...] + hsh[...]
    c = jnp.concatenate([left, right], axis=1) * dinv[...] + b[...]
    conv_ref[...] = c
    st = jnp.concatenate(
        [jnp.sum(c, axis=0, keepdims=True),
         jnp.sum(c * c, axis=0, keepdims=True)], axis=0)

    @pl.when(i == 0)
    def _():
        st_ref[...] = st

    @pl.when(i != 0)
    def _():
        st_ref[...] = st_ref[...] + st


def _tc2(plo, phi, hsl, hsh, dinv, b):
    half = pl.BlockSpec((RB, DH), lambda i: (i, 0))
    return pl.pallas_call(
        _tc2_body,
        grid=(G,),
        in_specs=[
            half, half, half, half,
            pl.BlockSpec((RB, 1), lambda i: (i, 0)),
            pl.BlockSpec((1, D), lambda i: (0, 0)),
        ],
        out_specs=[
            pl.BlockSpec((RB, D), lambda i: (i, 0)),
            pl.BlockSpec((2, D), lambda i: (0, 0)),
        ],
        out_shape=[
            jax.ShapeDtypeStruct((N, D), jnp.float32),
            jax.ShapeDtypeStruct((2, D), jnp.float32),
        ],
    )(plo, phi, hsl, hsh, dinv, b)


def _tc3_body(conv, st, res0, dinv, gw, gb, gms, w1,
              h1_ref, hsl_ref, hsh_ref):
    normed = _norm_apply(conv[...], st[...], gw[...], gb[...], gms[...])
    h1 = jnp.maximum(normed, 0.0) + res0[...]
    h1_ref[...] = h1
    hs1 = _dotT(h1, w1[...]) * dinv[...]
    hsl_ref[...] = hs1[:, :DH]
    hsh_ref[...] = hs1[:, DH:]


def _tc3(conv, st, res0, dinv, gw, gb, gms, w1):
    return pl.pallas_call(
        _tc3_body,
        grid=(G,),
        in_specs=[
            pl.BlockSpec((RB, D), lambda i: (i, 0)),
            pl.BlockSpec((2, D), lambda i: (0, 0)),
            pl.BlockSpec((RB, D), lambda i: (i, 0)),
            pl.BlockSpec((RB, 1), lambda i: (i, 0)),
            pl.BlockSpec((1, D), lambda i: (0, 0)),
            pl.BlockSpec((1, D), lambda i: (0, 0)),
            pl.BlockSpec((1, D), lambda i: (0, 0)),
            pl.BlockSpec((D, D), lambda i: (0, 0)),
        ],
        out_specs=[
            pl.BlockSpec((RB, D), lambda i: (i, 0)),
            pl.BlockSpec((RB, DH), lambda i: (i, 0)),
            pl.BlockSpec((RB, DH), lambda i: (i, 0)),
        ],
        out_shape=[
            jax.ShapeDtypeStruct((N, D), jnp.float32),
            jax.ShapeDtypeStruct((N, DH), jnp.float32),
            jax.ShapeDtypeStruct((N, DH), jnp.float32),
        ],
    )(conv, st, res0, dinv, gw, gb, gms, w1)


def _tc5_body(conv, st, h1, gw, gb, gms, l0w, l0b, l1w, l1b, tw, tb, out_ref):
    normed = _norm_apply(conv[...], st[...], gw[...], gb[...], gms[...])
    h2 = jnp.maximum(normed, 0.0) + h1[...]
    m = jnp.maximum(_dotT(h2, l0w[...]) + l0b[...], 0.0)
    m = jnp.maximum(_dotT(m, l1w[...]) + l1b[...], 0.0)
    out_ref[...] = _dotT(m, tw[...]) + tb[...]


def _tc5(conv, st, h1, gw, gb, gms, l0w, l0b, l1w, l1b, tw, tb):
    full = pl.BlockSpec((D, D), lambda i: (0, 0))
    vec = pl.BlockSpec((1, D), lambda i: (0, 0))
    row = pl.BlockSpec((RB, D), lambda i: (i, 0))
    return pl.pallas_call(
        _tc5_body,
        grid=(G,),
        in_specs=[row, pl.BlockSpec((2, D), lambda i: (0, 0)), row,
                  vec, vec, vec, full, vec, full, vec, full, vec],
        out_specs=row,
        out_shape=jax.ShapeDtypeStruct((N, D), jnp.float32),
    )(conv, st, h1, gw, gb, gms, l0w, l0b, l1w, l1b, tw, tb)


def kernel(x, edge_index, W0, b0, W1, b1, gn0_w, gn0_b, gn0_ms,
           gn1_w, gn1_b, gn1_ms, P, L0_W, L0_b, L1_W, L1_b, T_W, T_b):
    ei = edge_index.astype(jnp.int32)
    srcr = ei[0].reshape(NW, NCHUNK, CHUNK)
    dstr = ei[1].reshape(NW, NCHUNK, CHUNK)
    srcr2 = ei[0].reshape(NS, NCHUNK2, CHUNK)
    dstr2 = ei[1].reshape(NS, NCHUNK2, CHUNK)

    sc_deg, sc_edge = _sc_kernels()
    degp = sc_deg(dstr)
    d0 = degp[0, :N, 0:1]
    d1 = degp[1, :N, 0:1]

    hsl0, hsh0, res0, dinv = _tc1(x, d0, d1, W0, P)

    pa = sc_edge(hsl0, hsh0, srcr2, dstr2)
    conv0, st0 = _tc2(pa[0, :N], pa[1, :N],
                      hsl0, hsh0, dinv, b0.reshape(1, D))
    h1, hsl1, hsh1 = _tc3(conv0, st0, res0, dinv, gn0_w.reshape(1, D),
                          gn0_b.reshape(1, D), gn0_ms.reshape(1, D), W1)

    pb = sc_edge(hsl1, hsh1, srcr2, dstr2)
    conv1, st1 = _tc2(pb[0, :N], pb[1, :N],
                      hsl1, hsh1, dinv, b1.reshape(1, D))
    return _tc5(conv1, st1, h1, gn1_w.reshape(1, D), gn1_b.reshape(1, D),
                gn1_ms.reshape(1, D), L0_W, L0_b.reshape(1, D),
                L1_W, L1_b.reshape(1, D), T_W, T_b.reshape(1, D))


# speedup vs baseline: 1.1603x; 1.0113x over previous
import functools

import jax
import jax.numpy as jnp
from jax import lax
from jax.experimental import pallas as pl
from jax.experimental.pallas import tpu as pltpu
from jax.experimental.pallas import tpu_sc as plsc

N = 10000
E = 320000
D = 128
NC = 2
NS = 16
NW = NC * NS
CHUNK = 125
NCHUNK = (E // NW) // CHUNK
NCHUNK2 = (E // NS) // CHUNK
NBUF = 5
N_PAD = 10240
ROWS_SUB = N_PAD // NS
ZROWS = 128
DH = D // 2
DEG_PAD = 10240
DEG_SUB = DEG_PAD // NS
RB = 1000
G = N // RB
EPS = 1e-5

def _sc_deg_body(dstr, out, dst_v, ones_v, zb, acc, sem):
    del sem
    cid = lax.axis_index("c")
    sid = lax.axis_index("s")
    wid = sid * NC + cid

    @pl.loop(0, CHUNK)
    def _(i):
        ones_v[i, :] = jnp.ones((16,), jnp.float32)

    @pl.loop(0, DEG_SUB)
    def _(i):
        zb[i, :] = jnp.zeros((16,), jnp.float32)

    pltpu.sync_copy(dstr.at[wid], dst_v)
    pltpu.sync_copy(zb, acc.at[pl.ds(sid * DEG_SUB, DEG_SUB)])
    plsc.subcore_barrier()

    @pl.loop(0, NCHUNK)
    def _(j):
        pltpu.sync_copy(ones_v, acc.at[dst_v.at[j]], add=True)

    plsc.subcore_barrier()
    pltpu.sync_copy(acc.at[pl.ds(sid * DEG_SUB, DEG_SUB)],
                    out.at[cid, pl.ds(sid * DEG_SUB, DEG_SUB)])


def _sc_edge_body(hs_lo, hs_hi, srcr, dstr, out,
                  src_v, dst_v, rows, zbuf, acc, gsems, ssems):
    cid = lax.axis_index("c")
    sid = lax.axis_index("s")

    @pl.loop(0, ZROWS)
    def _(i):
        @pl.loop(0, DH // 16)
        def _(j):
            zbuf[i, pl.ds(j * 16, 16)] = jnp.zeros((16,), jnp.float32)

    pltpu.sync_copy(srcr.at[sid], src_v)
    pltpu.sync_copy(dstr.at[sid], dst_v)
    for k in range(ROWS_SUB // ZROWS):
        pltpu.sync_copy(
            zbuf, acc.at[pl.ds(sid * ROWS_SUB + k * ZROWS, ZROWS)])
    plsc.subcore_barrier()

    for half, tbl in enumerate((hs_lo, hs_hi)):
        @pl.when(cid == half)
        def _(tbl=tbl):
            for b in range(NBUF):
                pltpu.async_copy(tbl.at[src_v.at[b]], rows[b], gsems[b])

            @pl.loop(0, NCHUNK2, step=NBUF)
            def _(j):
                for b in range(NBUF):
                    pltpu.make_async_copy(
                        tbl.at[src_v.at[j + b]], rows[b], gsems[b]).wait()
                    pltpu.async_copy(
                        rows[b], acc.at[dst_v.at[j + b]], ssems[b], add=True)
                for b in range(NBUF):
                    pltpu.make_async_copy(
                        rows[b], acc.at[dst_v.at[j + b]], ssems[b]).wait()

                    @pl.when(j + b + NBUF < NCHUNK2)
                    def _():
                        pltpu.async_copy(
                            tbl.at[src_v.at[j + b + NBUF]], rows[b], gsems[b])

    plsc.subcore_barrier()
    pltpu.sync_copy(acc.at[pl.ds(sid * ROWS_SUB, ROWS_SUB)],
                    out.at[cid, pl.ds(sid * ROWS_SUB, ROWS_SUB)])


@functools.lru_cache(maxsize=None)
def _sc_kernels():
    mesh = plsc.VectorSubcoreMesh(
        core_axis_name="c", subcore_axis_name="s",
        num_cores=NC, num_subcores=NS)
    sc_deg = pl.kernel(
        _sc_deg_body,
        out_type=jax.ShapeDtypeStruct((NC, DEG_PAD, 16), jnp.float32),
        mesh=mesh,
        compiler_params=pltpu.CompilerParams(use_tc_tiling_on_sc=False),
        scratch_types=[
            pltpu.VMEM((NCHUNK, CHUNK), jnp.int32),
            pltpu.VMEM((CHUNK, 16), jnp.float32),
            pltpu.VMEM((DEG_SUB, 16), jnp.float32),
            pltpu.VMEM_SHARED((DEG_PAD, 16), jnp.float32),
            pltpu.SemaphoreType.DMA,
        ],
    )
    sc_edge = pl.kernel(
        _sc_edge_body,
        out_type=jax.ShapeDtypeStruct((NC, N_PAD, DH), jnp.float32),
        mesh=mesh,
        compiler_params=pltpu.CompilerParams(use_tc_tiling_on_sc=False),
        scratch_types=[
            pltpu.VMEM((NCHUNK2, CHUNK), jnp.int32),
            pltpu.VMEM((NCHUNK2, CHUNK), jnp.int32),
            [pltpu.VMEM((CHUNK, DH), jnp.float32) for _ in range(NBUF)],
            pltpu.VMEM((ZROWS, DH), jnp.float32),
            pltpu.VMEM_SHARED((N_PAD, DH), jnp.float32),
            [pltpu.SemaphoreType.DMA for _ in range(NBUF)],
            [pltpu.SemaphoreType.DMA for _ in range(NBUF)],
        ],
    )
    return sc_deg, sc_edge


def _dotT(a, w):
    return lax.dot_general(a, w, (((1,), (1,)), ((), ())),
                           preferred_element_type=jnp.float32)


def _tc1_body(x_ref, d0_ref, d1_ref, w0_ref, p_ref,
              hsl_ref, hsh_ref, res_ref, dinv_ref):
    deg = d0_ref[...] + d1_ref[...] + 1.0
    dinv = lax.rsqrt(deg)
    xb = x_ref[...]
    hs = _dotT(xb, w0_ref[...]) * dinv
    hsl_ref[...] = hs[:, :DH]
    hsh_ref[...] = hs[:, DH:]
    res_ref[...] = _dotT(xb, p_ref[...])
    dinv_ref[...] = dinv


def _tc1(x, d0, d1, w0, p):
    return pl.pallas_call(
        _tc1_body,
        grid=(G,),
        in_specs=[
            pl.BlockSpec((RB, D), lambda i: (i, 0)),
            pl.BlockSpec((RB, 1), lambda i: (i, 0)),
            pl.BlockSpec((RB, 1), lambda i: (i, 0)),
            pl.BlockSpec((D, D), lambda i: (0, 0)),
            pl.BlockSpec((D, D), lambda i: (0, 0)),
        ],
        out_specs=[
            pl.BlockSpec((RB, DH), lambda i: (i, 0)),
            pl.BlockSpec((RB, DH), lambda i: (i, 0)),
            pl.BlockSpec((RB, D), lambda i: (i, 0)),
            pl.BlockSpec((RB, 1), lambda i: (i, 0)),
        ],
        out_shape=[
            jax.ShapeDtypeStruct((N, DH), jnp.float32),
            jax.ShapeDtypeStruct((N, DH), jnp.float32),
            jax.ShapeDtypeStruct((N, D), jnp.float32),
            jax.ShapeDtypeStruct((N, 1), jnp.float32),
        ],
    )(x, d0, d1, w0, p)


def _norm_apply(c, stv, gw, gb, gms):
    mean = stv[0:1] / N
    ex2 = stv[1:2] / N
    var = ex2 - (2.0 * gms - gms * gms) * mean * mean
    inv = lax.rsqrt(var + EPS)
    return gw * (c - gms * mean) * inv + gb


def _tc2_body(plo, phi, hsl, hsh, dinv, b, conv_ref, st_ref):
    i = pl.program_id(0)
    left = plo[...] + hsl[...]
    right = phi[...] + hsh[...]
    c = jnp.concatenate([left, right], axis=1) * dinv[...] + b[...]
    conv_ref[...] = c
    st = jnp.concatenate(
        [jnp.sum(c, axis=0, keepdims=True),
         jnp.sum(c * c, axis=0, keepdims=True)], axis=0)

    @pl.when(i == 0)
    def _():
        st_ref[...] = st

    @pl.when(i != 0)
    def _():
        st_ref[...] = st_ref[...] + st


def _tc2(plo, phi, hsl, hsh, dinv, b):
    half = pl.BlockSpec((RB, DH), lambda i: (i, 0))
    return pl.pallas_call(
        _tc2_body,
        grid=(G,),
        in_specs=[
            half, half, half, half,
            pl.BlockSpec((RB, 1), lambda i: (i, 0)),
            pl.BlockSpec((1, D), lambda i: (0, 0)),
        ],
        out_specs=[
            pl.BlockSpec((RB, D), lambda i: (i, 0)),
            pl.BlockSpec((2, D), lambda i: (0, 0)),
        ],
        out_shape=[
            jax.ShapeDtypeStruct((N, D), jnp.float32),
            jax.ShapeDtypeStruct((2, D), jnp.float32),
        ],
    )(plo, phi, hsl, hsh, dinv, b)


def _tc3_body(conv, st, res0, dinv, gw, gb, gms, w1,
              h1_ref, hsl_ref, hsh_ref):
    normed = _norm_apply(conv[...], st[...], gw[...], gb[...], gms[...])
    h1 = jnp.maximum(normed, 0.0) + res0[...]
    h1_ref[...] = h1
    hs1 = _dotT(h1, w1[...]) * dinv[...]
    hsl_ref[...] = hs1[:, :DH]
    hsh_ref[...] = hs1[:, DH:]


def _tc3(conv, st, res0, dinv, gw, gb, gms, w1):
    return pl.pallas_call(
        _tc3_body,
        grid=(G,),
        in_specs=[
            pl.BlockSpec((RB, D), lambda i: (i, 0)),
            pl.BlockSpec((2, D), lambda i: (0, 0)),
            pl.BlockSpec((RB, D), lambda i: (i, 0)),
            pl.BlockSpec((RB, 1), lambda i: (i, 0)),
            pl.BlockSpec((1, D), lambda i: (0, 0)),
            pl.BlockSpec((1, D), lambda i: (0, 0)),
            pl.BlockSpec((1, D), lambda i: (0, 0)),
            pl.BlockSpec((D, D), lambda i: (0, 0)),
        ],
        out_specs=[
            pl.BlockSpec((RB, D), lambda i: (i, 0)),
            pl.BlockSpec((RB, DH), lambda i: (i, 0)),
            pl.BlockSpec((RB, DH), lambda i: (i, 0)),
        ],
        out_shape=[
            jax.ShapeDtypeStruct((N, D), jnp.float32),
            jax.ShapeDtypeStruct((N, DH), jnp.float32),
            jax.ShapeDtypeStruct((N, DH), jnp.float32),
        ],
    )(conv, st, res0, dinv, gw, gb, gms, w1)


def _tc5_body(conv, st, h1, gw, gb, gms, l0w, l0b, l1w, l1b, tw, tb, out_ref):
    normed = _norm_apply(conv[...], st[...], gw[...], gb[...], gms[...])
    h2 = jnp.maximum(normed, 0.0) + h1[...]
    m = jnp.maximum(_dotT(h2, l0w[...]) + l0b[...], 0.0)
    m = jnp.maximum(_dotT(m, l1w[...]) + l1b[...], 0.0)
    out_ref[...] = _dotT(m, tw[...]) + tb[...]


def _tc5(conv, st, h1, gw, gb, gms, l0w, l0b, l1w, l1b, tw, tb):
    full = pl.BlockSpec((D, D), lambda i: (0, 0))
    vec = pl.BlockSpec((1, D), lambda i: (0, 0))
    row = pl.BlockSpec((RB, D), lambda i: (i, 0))
    return pl.pallas_call(
        _tc5_body,
        grid=(G,),
        in_specs=[row, pl.BlockSpec((2, D), lambda i: (0, 0)), row,
                  vec, vec, vec, full, vec, full, vec, full, vec],
        out_specs=row,
        out_shape=jax.ShapeDtypeStruct((N, D), jnp.float32),
    )(conv, st, h1, gw, gb, gms, l0w, l0b, l1w, l1b, tw, tb)


def kernel(x, edge_index, W0, b0, W1, b1, gn0_w, gn0_b, gn0_ms,
           gn1_w, gn1_b, gn1_ms, P, L0_W, L0_b, L1_W, L1_b, T_W, T_b):
    ei = edge_index.astype(jnp.int32)
    srcr = ei[0].reshape(NW, NCHUNK, CHUNK)
    dstr = ei[1].reshape(NW, NCHUNK, CHUNK)
    srcr2 = ei[0].reshape(NS, NCHUNK2, CHUNK)
    dstr2 = ei[1].reshape(NS, NCHUNK2, CHUNK)

    sc_deg, sc_edge = _sc_kernels()
    degp = sc_deg(dstr)
    d0 = degp[0, :N, 0:1]
    d1 = degp[1, :N, 0:1]

    hsl0, hsh0, res0, dinv = _tc1(x, d0, d1, W0, P)

    pa = sc_edge(hsl0, hsh0, srcr2, dstr2)
    conv0, st0 = _tc2(pa[0, :N], pa[1, :N],
                      hsl0, hsh0, dinv, b0.reshape(1, D))
    h1, hsl1, hsh1 = _tc3(conv0, st0, res0, dinv, gn0_w.reshape(1, D),
                          gn0_b.reshape(1, D), gn0_ms.reshape(1, D), W1)

    pb = sc_edge(hsl1, hsh1, srcr2, dstr2)
    conv1, st1 = _tc2(pb[0, :N], pb[1, :N],
                      hsl1, hsh1, dinv, b1.reshape(1, D))
    return _tc5(conv1, st1, h1, gn1_w.reshape(1, D), gn1_b.reshape(1, D),
                gn1_ms.reshape(1, D), L0_W, L0_b.reshape(1, D),
                L1_W, L1_b.reshape(1, D), T_W, T_b.reshape(1, D))


# speedup vs baseline: 1.2193x; 1.0508x over previous
import functools

import jax
import jax.numpy as jnp
from jax import lax
from jax.experimental import pallas as pl
from jax.experimental.pallas import tpu as pltpu
from jax.experimental.pallas import tpu_sc as plsc

N = 10000
E = 320000
D = 128
NC = 2
NS = 16
NW = NC * NS
CHUNK = 125
NCHUNK = (E // NW) // CHUNK
NCHUNK2 = (E // NS) // CHUNK
NBUF = 5
N_PAD = 10240
ROWS_SUB = N_PAD // NS
ZROWS = 128
DH = D // 2
DEG_PAD = 10240
DEG_SUB = DEG_PAD // NS
RB = 1000
G = N // RB
EPS = 1e-5

def _sc_deg_body(dstr, out, dst_v, ones_v, zb, acc, sem):
    del sem
    cid = lax.axis_index("c")
    sid = lax.axis_index("s")

    @pl.loop(0, CHUNK)
    def _(i):
        ones_v[i, :] = jnp.ones((16,), jnp.float32)

    @pl.loop(0, DEG_SUB)
    def _(i):
        zb[i, :] = jnp.zeros((16,), jnp.float32)

    pltpu.sync_copy(dstr.at[sid, pl.ds(cid * NCHUNK, NCHUNK)], dst_v)
    pltpu.sync_copy(zb, acc.at[pl.ds(sid * DEG_SUB, DEG_SUB)])
    plsc.subcore_barrier()

    @pl.loop(0, NCHUNK)
    def _(j):
        pltpu.sync_copy(ones_v, acc.at[dst_v.at[j]], add=True)

    plsc.subcore_barrier()
    pltpu.sync_copy(acc.at[pl.ds(sid * DEG_SUB, DEG_SUB)],
                    out.at[cid, pl.ds(sid * DEG_SUB, DEG_SUB)])


def _sc_edge_body(hs_lo, hs_hi, srcr, dstr, out,
                  src_v, dst_v, rows, zbuf, acc, gsems, ssems):
    cid = lax.axis_index("c")
    sid = lax.axis_index("s")

    @pl.loop(0, ZROWS)
    def _(i):
        @pl.loop(0, DH // 16)
        def _(j):
            zbuf[i, pl.ds(j * 16, 16)] = jnp.zeros((16,), jnp.float32)

    pltpu.sync_copy(srcr.at[sid], src_v)
    pltpu.sync_copy(dstr.at[sid], dst_v)
    for k in range(ROWS_SUB // ZROWS):
        pltpu.sync_copy(
            zbuf, acc.at[pl.ds(sid * ROWS_SUB + k * ZROWS, ZROWS)])
    plsc.subcore_barrier()

    for half, tbl in enumerate((hs_lo, hs_hi)):
        @pl.when(cid == half)
        def _(tbl=tbl):
            for b in range(NBUF):
                pltpu.async_copy(tbl.at[src_v.at[b]], rows[b], gsems[b])

            @pl.loop(0, NCHUNK2, step=NBUF)
            def _(j):
                for b in range(NBUF):
                    pltpu.make_async_copy(
                        tbl.at[src_v.at[j + b]], rows[b], gsems[b]).wait()
                    pltpu.async_copy(
                        rows[b], acc.at[dst_v.at[j + b]], ssems[b], add=True)
                for b in range(NBUF):
                    pltpu.make_async_copy(
                        rows[b], acc.at[dst_v.at[j + b]], ssems[b]).wait()

                    @pl.when(j + b + NBUF < NCHUNK2)
                    def _():
                        pltpu.async_copy(
                            tbl.at[src_v.at[j + b + NBUF]], rows[b], gsems[b])

    plsc.subcore_barrier()
    pltpu.sync_copy(acc.at[pl.ds(sid * ROWS_SUB, ROWS_SUB)],
                    out.at[cid, pl.ds(sid * ROWS_SUB, ROWS_SUB)])


@functools.lru_cache(maxsize=None)
def _sc_kernels():
    mesh = plsc.VectorSubcoreMesh(
        core_axis_name="c", subcore_axis_name="s",
        num_cores=NC, num_subcores=NS)
    sc_deg = pl.kernel(
        _sc_deg_body,
        out_type=jax.ShapeDtypeStruct((NC, DEG_PAD, 16), jnp.float32),
        mesh=mesh,
        compiler_params=pltpu.CompilerParams(use_tc_tiling_on_sc=False),
        scratch_types=[
            pltpu.VMEM((NCHUNK, CHUNK), jnp.int32),
            pltpu.VMEM((CHUNK, 16), jnp.float32),
            pltpu.VMEM((DEG_SUB, 16), jnp.float32),
            pltpu.VMEM_SHARED((DEG_PAD, 16), jnp.float32),
            pltpu.SemaphoreType.DMA,
        ],
    )
    sc_edge = pl.kernel(
        _sc_edge_body,
        out_type=jax.ShapeDtypeStruct((NC, N_PAD, DH), jnp.float32),
        mesh=mesh,
        compiler_params=pltpu.CompilerParams(use_tc_tiling_on_sc=False),
        scratch_types=[
            pltpu.VMEM((NCHUNK2, CHUNK), jnp.int32),
            pltpu.VMEM((NCHUNK2, CHUNK), jnp.int32),
            [pltpu.VMEM((CHUNK, DH), jnp.float32) for _ in range(NBUF)],
            pltpu.VMEM((ZROWS, DH), jnp.float32),
            pltpu.VMEM_SHARED((N_PAD, DH), jnp.float32),
            [pltpu.SemaphoreType.DMA for _ in range(NBUF)],
            [pltpu.SemaphoreType.DMA for _ in range(NBUF)],
        ],
    )
    return sc_deg, sc_edge


def _dotT(a, w):
    return lax.dot_general(a, w, (((1,), (1,)), ((), ())),
                           preferred_element_type=jnp.float32)


def _dinv_from(d0_ref, d1_ref):
    deg = d0_ref[0][:, 0:1] + d1_ref[0][:, 0:1] + 1.0
    return lax.rsqrt(deg)


_DEG0 = pl.BlockSpec((1, RB, 16), lambda i: (0, i, 0))
_DEG1 = pl.BlockSpec((1, RB, 16), lambda i: (1, i, 0))


def _tc1_body(x_ref, d0_ref, d1_ref, w0_ref, p_ref,
              hsl_ref, hsh_ref, res_ref):
    dinv = _dinv_from(d0_ref, d1_ref)
    xb = x_ref[...]
    hs = _dotT(xb, w0_ref[...]) * dinv
    hsl_ref[...] = hs[:, :DH]
    hsh_ref[...] = hs[:, DH:]
    res_ref[...] = _dotT(xb, p_ref[...])


def _tc1(x, degp, w0, p):
    return pl.pallas_call(
        _tc1_body,
        grid=(G,),
        in_specs=[
            pl.BlockSpec((RB, D), lambda i: (i, 0)),
            _DEG0,
            _DEG1,
            pl.BlockSpec((D, D), lambda i: (0, 0)),
            pl.BlockSpec((D, D), lambda i: (0, 0)),
        ],
        out_specs=[
            pl.BlockSpec((RB, DH), lambda i: (i, 0)),
            pl.BlockSpec((RB, DH), lambda i: (i, 0)),
            pl.BlockSpec((RB, D), lambda i: (i, 0)),
        ],
        out_shape=[
            jax.ShapeDtypeStruct((N, DH), jnp.float32),
            jax.ShapeDtypeStruct((N, DH), jnp.float32),
            jax.ShapeDtypeStruct((N, D), jnp.float32),
        ],
    )(x, degp, degp, w0, p)


def _norm_apply(c, stv, gw, gb, gms):
    mean = stv[0:1] / N
    ex2 = stv[1:2] / N
    var = ex2 - (2.0 * gms - gms * gms) * mean * mean
    inv = lax.rsqrt(var + EPS)
    return gw * (c - gms * mean) * inv + gb


def _tc2_body(plo, phi, hsl, hsh, d0_ref, d1_ref, b, conv_ref, st_ref):
    i = pl.program_id(0)
    dinv = _dinv_from(d0_ref, d1_ref)
    left = plo[0] + hsl[...]
    right = phi[0] + hsh[...]
    c = jnp.concatenate([left, right], axis=1) * dinv + b[...]
    conv_ref[...] = c
    st = jnp.concatenate(
        [jnp.sum(c, axis=0, keepdims=True),
         jnp.sum(c * c, axis=0, keepdims=True)], axis=0)

    @pl.when(i == 0)
    def _():
        st_ref[...] = st

    @pl.when(i != 0)
    def _():
        st_ref[...] = st_ref[...] + st


def _tc2(pp, hsl, hsh, degp, b):
    half = pl.BlockSpec((RB, DH), lambda i: (i, 0))
    phalf0 = pl.BlockSpec((1, RB, DH), lambda i: (0, i, 0))
    phalf1 = pl.BlockSpec((1, RB, DH), lambda i: (1, i, 0))
    return pl.pallas_call(
        _tc2_body,
        grid=(G,),
        in_specs=[
            phalf0, phalf1, half, half, _DEG0, _DEG1,
            pl.BlockSpec((1, D), lambda i: (0, 0)),
        ],
        out_specs=[
            pl.BlockSpec((RB, D), lambda i: (i, 0)),
            pl.BlockSpec((2, D), lambda i: (0, 0)),
        ],
        out_shape=[
            jax.ShapeDtypeStruct((N, D), jnp.float32),
            jax.ShapeDtypeStruct((2, D), jnp.float32),
        ],
    )(pp, pp, hsl, hsh, degp, degp, b)


def _tc3_body(conv, st, res0, d0_ref, d1_ref, gw, gb, gms, w1,
              h1_ref, hsl_ref, hsh_ref):
    dinv = _dinv_from(d0_ref, d1_ref)
    normed = _norm_apply(conv[...], st[...], gw[...], gb[...], gms[...])
    h1 = jnp.maximum(normed, 0.0) + res0[...]
    h1_ref[...] = h1
    hs1 = _dotT(h1, w1[...]) * dinv
    hsl_ref[...] = hs1[:, :DH]
    hsh_ref[...] = hs1[:, DH:]


def _tc3(conv, st, res0, degp, gw, gb, gms, w1):
    return pl.pallas_call(
        _tc3_body,
        grid=(G,),
        in_specs=[
            pl.BlockSpec((RB, D), lambda i: (i, 0)),
            pl.BlockSpec((2, D), lambda i: (0, 0)),
            pl.BlockSpec((RB, D), lambda i: (i, 0)),
            _DEG0,
            _DEG1,
            pl.BlockSpec((1, D), lambda i: (0, 0)),
            pl.BlockSpec((1, D), lambda i: (0, 0)),
            pl.BlockSpec((1, D), lambda i: (0, 0)),
            pl.BlockSpec((D, D), lambda i: (0, 0)),
        ],
        out_specs=[
            pl.BlockSpec((RB, D), lambda i: (i, 0)),
            pl.BlockSpec((RB, DH), lambda i: (i, 0)),
            pl.BlockSpec((RB, DH), lambda i: (i, 0)),
        ],
        out_shape=[
            jax.ShapeDtypeStruct((N, D), jnp.float32),
            jax.ShapeDtypeStruct((N, DH), jnp.float32),
            jax.ShapeDtypeStruct((N, DH), jnp.float32),
        ],
    )(conv, st, res0, degp, degp, gw, gb, gms, w1)


def _tc5_body(conv, st, h1, gw, gb, gms, l0w, l0b, l1w, l1b, tw, tb, out_ref):
    normed = _norm_apply(conv[...], st[...], gw[...], gb[...], gms[...])
    h2 = jnp.maximum(normed, 0.0) + h1[...]
    m = jnp.maximum(_dotT(h2, l0w[...]) + l0b[...], 0.0)
    m = jnp.maximum(_dotT(m, l1w[...]) + l1b[...], 0.0)
    out_ref[...] = _dotT(m, tw[...]) + tb[...]


def _tc5(conv, st, h1, gw, gb, gms, l0w, l0b, l1w, l1b, tw, tb):
    full = pl.BlockSpec((D, D), lambda i: (0, 0))
    vec = pl.BlockSpec((1, D), lambda i: (0, 0))
    row = pl.BlockSpec((RB, D), lambda i: (i, 0))
    return pl.pallas_call(
        _tc5_body,
        grid=(G,),
        in_specs=[row, pl.BlockSpec((2, D), lambda i: (0, 0)), row,
                  vec, vec, vec, full, vec, full, vec, full, vec],
        out_specs=row,
        out_shape=jax.ShapeDtypeStruct((N, D), jnp.float32),
    )(conv, st, h1, gw, gb, gms, l0w, l0b, l1w, l1b, tw, tb)


def kernel(x, edge_index, W0, b0, W1, b1, gn0_w, gn0_b, gn0_ms,
           gn1_w, gn1_b, gn1_ms, P, L0_W, L0_b, L1_W, L1_b, T_W, T_b):
    ei = edge_index.astype(jnp.int32)
    srcr2 = ei[0].reshape(NS, NCHUNK2, CHUNK)
    dstr2 = ei[1].reshape(NS, NCHUNK2, CHUNK)

    sc_deg, sc_edge = _sc_kernels()
    degp = sc_deg(dstr2)

    hsl0, hsh0, res0 = _tc1(x, degp, W0, P)

    pa = sc_edge(hsl0, hsh0, srcr2, dstr2)
    conv0, st0 = _tc2(pa, hsl0, hsh0, degp, b0.reshape(1, D))
    h1, hsl1, hsh1 = _tc3(conv0, st0, res0, degp, gn0_w.reshape(1, D),
                          gn0_b.reshape(1, D), gn0_ms.reshape(1, D), W1)

    pb = sc_edge(hsl1, hsh1, srcr2, dstr2)
    conv1, st1 = _tc2(pb, hsl1, hsh1, degp, b1.reshape(1, D))
    return _tc5(conv1, st1, h1, gn1_w.reshape(1, D), gn1_b.reshape(1, D),
                gn1_ms.reshape(1, D), L0_W, L0_b.reshape(1, D),
                L1_W, L1_b.reshape(1, D), T_W, T_b.reshape(1, D))


# speedup vs baseline: 1.2824x; 1.0518x over previous
import functools

import jax
import jax.numpy as jnp
from jax import lax
from jax.experimental import pallas as pl
from jax.experimental.pallas import tpu as pltpu
from jax.experimental.pallas import tpu_sc as plsc

N = 10000
E = 320000
D = 128
NC = 2
NS = 16
NW = NC * NS
CHUNK = 125
NCHUNK = (E // NW) // CHUNK
NCHUNK2 = (E // NS) // CHUNK
NBUF = 5
N_PAD = 10240
ROWS_SUB = N_PAD // NS
ZROWS = 128
DH = D // 2
DEG_PAD = 10240
DEG_SUB = DEG_PAD // NS
RB = 1000
G = N // RB
EPS = 1e-5

def _sc_deg_body(dstr, out, dst_v, ones_v, zb, acc, sem):
    del sem
    cid = lax.axis_index("c")
    sid = lax.axis_index("s")

    @pl.loop(0, CHUNK)
    def _(i):
        ones_v[i, :] = jnp.ones((16,), jnp.float32)

    @pl.loop(0, DEG_SUB)
    def _(i):
        zb[i, :] = jnp.zeros((16,), jnp.float32)

    pltpu.sync_copy(dstr.at[sid, pl.ds(cid * NCHUNK, NCHUNK)], dst_v)
    pltpu.sync_copy(zb, acc.at[pl.ds(sid * DEG_SUB, DEG_SUB)])
    plsc.subcore_barrier()

    @pl.loop(0, NCHUNK)
    def _(j):
        pltpu.sync_copy(ones_v, acc.at[dst_v.at[j]], add=True)

    plsc.subcore_barrier()
    pltpu.sync_copy(acc.at[pl.ds(sid * DEG_SUB, DEG_SUB)],
                    out.at[cid, pl.ds(sid * DEG_SUB, DEG_SUB)])


def _sc_edge_body(hs_lo, hs_hi, srcr, dstr, out,
                  src_v, dst_v, rows, zbuf, acc, gsems, ssems):
    cid = lax.axis_index("c")
    sid = lax.axis_index("s")

    @pl.loop(0, ZROWS)
    def _(i):
        @pl.loop(0, DH // 16)
        def _(j):
            zbuf[i, pl.ds(j * 16, 16)] = jnp.zeros((16,), jnp.float32)

    pltpu.sync_copy(srcr.at[sid], src_v)
    pltpu.sync_copy(dstr.at[sid], dst_v)
    for k in range(ROWS_SUB // ZROWS):
        pltpu.sync_copy(
            zbuf, acc.at[pl.ds(sid * ROWS_SUB + k * ZROWS, ZROWS)])
    plsc.subcore_barrier()

    for half, tbl in enumerate((hs_lo, hs_hi)):
        @pl.when(cid == half)
        def _(tbl=tbl):
            for b in range(NBUF):
                pltpu.async_copy(tbl.at[src_v.at[b]], rows[b], gsems[b])

            @pl.loop(0, NCHUNK2, step=NBUF)
            def _(j):
                for b in range(NBUF):
                    pltpu.make_async_copy(
                        tbl.at[src_v.at[j + b]], rows[b], gsems[b]).wait()
                    pltpu.async_copy(
                        rows[b], acc.at[dst_v.at[j + b]], ssems[b], add=True)
                for b in range(NBUF):
                    pltpu.make_async_copy(
                        rows[b], acc.at[dst_v.at[j + b]], ssems[b]).wait()

                    @pl.when(j + b + NBUF < NCHUNK2)
                    def _():
                        pltpu.async_copy(
                            tbl.at[src_v.at[j + b + NBUF]], rows[b], gsems[b])

    plsc.subcore_barrier()
    pltpu.sync_copy(acc.at[pl.ds(sid * ROWS_SUB, ROWS_SUB)],
                    out.at[cid, pl.ds(sid * ROWS_SUB, ROWS_SUB),
                           pl.ds(cid * DH, DH)])


@functools.lru_cache(maxsize=None)
def _sc_kernels():
    mesh = plsc.VectorSubcoreMesh(
        core_axis_name="c", subcore_axis_name="s",
        num_cores=NC, num_subcores=NS)
    sc_deg = pl.kernel(
        _sc_deg_body,
        out_type=jax.ShapeDtypeStruct((NC, DEG_PAD, 16), jnp.float32),
        mesh=mesh,
        compiler_params=pltpu.CompilerParams(use_tc_tiling_on_sc=False),
        scratch_types=[
            pltpu.VMEM((NCHUNK, CHUNK), jnp.int32),
            pltpu.VMEM((CHUNK, 16), jnp.float32),
            pltpu.VMEM((DEG_SUB, 16), jnp.float32),
            pltpu.VMEM_SHARED((DEG_PAD, 16), jnp.float32),
            pltpu.SemaphoreType.DMA,
        ],
    )
    sc_edge = pl.kernel(
        _sc_edge_body,
        out_type=jax.ShapeDtypeStruct((NC, N_PAD, D), jnp.float32),
        mesh=mesh,
        compiler_params=pltpu.CompilerParams(use_tc_tiling_on_sc=False),
        scratch_types=[
            pltpu.VMEM((NCHUNK2, CHUNK), jnp.int32),
            pltpu.VMEM((NCHUNK2, CHUNK), jnp.int32),
            [pltpu.VMEM((CHUNK, DH), jnp.float32) for _ in range(NBUF)],
            pltpu.VMEM((ZROWS, DH), jnp.float32),
            pltpu.VMEM_SHARED((N_PAD, DH), jnp.float32),
            [pltpu.SemaphoreType.DMA for _ in range(NBUF)],
            [pltpu.SemaphoreType.DMA for _ in range(NBUF)],
        ],
    )
    return sc_deg, sc_edge


def _dotT(a, w):
    return lax.dot_general(a, w, (((1,), (1,)), ((), ())),
                           preferred_element_type=jnp.float32)


def _dinv_from(d0_ref, d1_ref):
    deg = d0_ref[0][:, 0:1] + d1_ref[0][:, 0:1] + 1.0
    return lax.rsqrt(deg)


_DEG0 = pl.BlockSpec((1, RB, 16), lambda i: (0, i, 0))
_DEG1 = pl.BlockSpec((1, RB, 16), lambda i: (1, i, 0))


def _tc1_body(x_ref, d0_ref, d1_ref, w0_ref, p_ref,
              hsl_ref, hsh_ref, res_ref):
    dinv = _dinv_from(d0_ref, d1_ref)
    xb = x_ref[...]
    hs = _dotT(xb, w0_ref[...]) * dinv
    hsl_ref[...] = hs[:, :DH]
    hsh_ref[...] = hs[:, DH:]
    res_ref[...] = _dotT(xb, p_ref[...])


def _tc1(x, degp, w0, p):
    return pl.pallas_call(
        _tc1_body,
        grid=(G,),
        in_specs=[
            pl.BlockSpec((RB, D), lambda i: (i, 0)),
            _DEG0,
            _DEG1,
            pl.BlockSpec((D, D), lambda i: (0, 0)),
            pl.BlockSpec((D, D), lambda i: (0, 0)),
        ],
        out_specs=[
            pl.BlockSpec((RB, DH), lambda i: (i, 0)),
            pl.BlockSpec((RB, DH), lambda i: (i, 0)),
            pl.BlockSpec((RB, D), lambda i: (i, 0)),
        ],
        out_shape=[
            jax.ShapeDtypeStruct((N, DH), jnp.float32),
            jax.ShapeDtypeStruct((N, DH), jnp.float32),
            jax.ShapeDtypeStruct((N, D), jnp.float32),
        ],
    )(x, degp, degp, w0, p)


def _norm_apply(c, stv, gw, gb, gms):
    mean = stv[0:1] / N
    ex2 = stv[1:2] / N
    var = ex2 - (2.0 * gms - gms * gms) * mean * mean
    inv = lax.rsqrt(var + EPS)
    return gw * (c - gms * mean) * inv + gb


def _tc2_body(plo, phi, hsl, hsh, d0_ref, d1_ref, b, conv_ref, st_ref):
    i = pl.program_id(0)
    dinv = _dinv_from(d0_ref, d1_ref)
    left = plo[0][:, :DH] + hsl[...]
    right = phi[0][:, DH:] + hsh[...]
    c = jnp.concatenate([left, right], axis=1) * dinv + b[...]
    conv_ref[...] = c
    st = jnp.concatenate(
        [jnp.sum(c, axis=0, keepdims=True),
         jnp.sum(c * c, axis=0, keepdims=True)], axis=0)

    @pl.when(i == 0)
    def _():
        st_ref[...] = st

    @pl.when(i != 0)
    def _():
        st_ref[...] = st_ref[...] + st


def _tc2(pp, hsl, hsh, degp, b):
    half = pl.BlockSpec((RB, DH), lambda i: (i, 0))
    phalf0 = pl.BlockSpec((1, RB, D), lambda i: (0, i, 0))
    phalf1 = pl.BlockSpec((1, RB, D), lambda i: (1, i, 0))
    return pl.pallas_call(
        _tc2_body,
        grid=(G,),
        in_specs=[
            phalf0, phalf1, half, half, _DEG0, _DEG1,
            pl.BlockSpec((1, D), lambda i: (0, 0)),
        ],
        out_specs=[
            pl.BlockSpec((RB, D), lambda i: (i, 0)),
            pl.BlockSpec((2, D), lambda i: (0, 0)),
        ],
        out_shape=[
            jax.ShapeDtypeStruct((N, D), jnp.float32),
            jax.ShapeDtypeStruct((2, D), jnp.float32),
        ],
    )(pp, pp, hsl, hsh, degp, degp, b)


def _tc3_body(conv, st, res0, d0_ref, d1_ref, gw, gb, gms, w1,
              h1_ref, hsl_ref, hsh_ref):
    dinv = _dinv_from(d0_ref, d1_ref)
    normed = _norm_apply(conv[...], st[...], gw[...], gb[...], gms[...])
    h1 = jnp.maximum(normed, 0.0) + res0[...]
    h1_ref[...] = h1
    hs1 = _dotT(h1, w1[...]) * dinv
    hsl_ref[...] = hs1[:, :DH]
    hsh_ref[...] = hs1[:, DH:]


def _tc3(conv, st, res0, degp, gw, gb, gms, w1):
    return pl.pallas_call(
        _tc3_body,
        grid=(G,),
        in_specs=[
            pl.BlockSpec((RB, D), lambda i: (i, 0)),
            pl.BlockSpec((2, D), lambda i: (0, 0)),
            pl.BlockSpec((RB, D), lambda i: (i, 0)),
            _DEG0,
            _DEG1,
            pl.BlockSpec((1, D), lambda i: (0, 0)),
            pl.BlockSpec((1, D), lambda i: (0, 0)),
            pl.BlockSpec((1, D), lambda i: (0, 0)),
            pl.BlockSpec((D, D), lambda i: (0, 0)),
        ],
        out_specs=[
            pl.BlockSpec((RB, D), lambda i: (i, 0)),
            pl.BlockSpec((RB, DH), lambda i: (i, 0)),
            pl.BlockSpec((RB, DH), lambda i: (i, 0)),
        ],
        out_shape=[
            jax.ShapeDtypeStruct((N, D), jnp.float32),
            jax.ShapeDtypeStruct((N, DH), jnp.float32),
            jax.ShapeDtypeStruct((N, DH), jnp.float32),
        ],
    )(conv, st, res0, degp, degp, gw, gb, gms, w1)


def _tc5_body(conv, st, h1, gw, gb, gms, l0w, l0b, l1w, l1b, tw, tb, out_ref):
    normed = _norm_apply(conv[...], st[...], gw[...], gb[...], gms[...])
    h2 = jnp.maximum(normed, 0.0) + h1[...]
    m = jnp.maximum(_dotT(h2, l0w[...]) + l0b[...], 0.0)
    m = jnp.maximum(_dotT(m, l1w[...]) + l1b[...], 0.0)
    out_ref[...] = _dotT(m, tw[...]) + tb[...]


def _tc5(conv, st, h1, gw, gb, gms, l0w, l0b, l1w, l1b, tw, tb):
    full = pl.BlockSpec((D, D), lambda i: (0, 0))
    vec = pl.BlockSpec((1, D), lambda i: (0, 0))
    row = pl.BlockSpec((RB, D), lambda i: (i, 0))
    return pl.pallas_call(
        _tc5_body,
        grid=(G,),
        in_specs=[row, pl.BlockSpec((2, D), lambda i: (0, 0)), row,
                  vec, vec, vec, full, vec, full, vec, full, vec],
        out_specs=row,
        out_shape=jax.ShapeDtypeStruct((N, D), jnp.float32),
    )(conv, st, h1, gw, gb, gms, l0w, l0b, l1w, l1b, tw, tb)


def kernel(x, edge_index, W0, b0, W1, b1, gn0_w, gn0_b, gn0_ms,
           gn1_w, gn1_b, gn1_ms, P, L0_W, L0_b, L1_W, L1_b, T_W, T_b):
    ei = edge_index.astype(jnp.int32)
    srcr2 = ei[0].reshape(NS, NCHUNK2, CHUNK)
    dstr2 = ei[1].reshape(NS, NCHUNK2, CHUNK)

    sc_deg, sc_edge = _sc_kernels()
    degp = sc_deg(dstr2)

    hsl0, hsh0, res0 = _tc1(x, degp, W0, P)

    pa = sc_edge(hsl0, hsh0, srcr2, dstr2)
    conv0, st0 = _tc2(pa, hsl0, hsh0, degp, b0.reshape(1, D))
    h1, hsl1, hsh1 = _tc3(conv0, st0, res0, degp, gn0_w.reshape(1, D),
                          gn0_b.reshape(1, D), gn0_ms.reshape(1, D), W1)

    pb = sc_edge(hsl1, hsh1, srcr2, dstr2)
    conv1, st1 = _tc2(pb, hsl1, hsh1, degp, b1.reshape(1, D))
    return _tc5(conv1, st1, h1, gn1_w.reshape(1, D), gn1_b.reshape(1, D),
                gn1_ms.reshape(1, D), L0_W, L0_b.reshape(1, D),
                L1_W, L1_b.reshape(1, D), T_W, T_b.reshape(1, D))
